# baseline (speedup 1.0000x reference)
import functools
import jax
import jax.numpy as jnp
from jax import lax
from jax.experimental import pallas as pl
from jax.experimental.pallas import tpu as pltpu, tpu_sc as plsc

N = 10000
E = 160000
D = 128
H = 8
C1 = 512
C2 = 256
G = 64
OUT = 10

NW = 32
NPW = 320
N_PAD = NW * NPW
E2_PAD = E + 16 * N_PAD + 64


def _mm_body(x_ref, w_ref, b_ref, o_ref):
    o_ref[...] = jnp.dot(x_ref[...], w_ref[...],
                         preferred_element_type=jnp.float32) + b_ref[...][None, :]


def _matmul_bias(x, w, b, block_n=1024, block_c=512):
    n, din = x.shape
    dout = w.shape[1]
    bc = min(block_c, dout)
    grid = (n // block_n, dout // bc)
    return pl.pallas_call(
        _mm_body,
        grid=grid,
        in_specs=[
            pl.BlockSpec((block_n, din), lambda i, j: (i, 0)),
            pl.BlockSpec((din, bc), lambda i, j: (0, j)),
            pl.BlockSpec((bc,), lambda i, j: (j,)),
        ],
        out_specs=pl.BlockSpec((block_n, bc), lambda i, j: (i, j)),
        out_shape=jax.ShapeDtypeStruct((n, dout), jnp.float32),
    )(x, w, b)


def _perm(x, idx):
    dnums = lax.GatherDimensionNumbers(
        offset_dims=(), collapsed_slice_dims=(0,), start_index_map=(0,))
    return lax.gather(x, idx[:, None], dnums, (1,),
                      mode=lax.GatherScatterMode.PROMISE_IN_BOUNDS)


def _lane_sum_splat(x):
    lane = lax.broadcasted_iota(jnp.int32, (16,), 0)
    for sh in (8, 4, 2, 1):
        x = x + _perm(x, lane ^ sh)
    return x


def _get_scalar_i32(ref, i):
    g = (i // 16) * 16
    vec = ref[pl.ds(g, 16)].astype(jnp.float32)
    lane = lax.broadcasted_iota(jnp.int32, (16,), 0)
    s = _lane_sum_splat(jnp.where(lane == (i - g), vec, 0.0))
    return jnp.squeeze(lax.slice(s, (0,), (1,))).astype(jnp.int32)


def _make_sc_edge_kernel(hc, c, grp):
    rs = 1.0 / (c ** 0.5)
    nch = c // 16

    def body(q_hbm, k_hbm, v_hbm, srcp_hbm, iptr2_hbm, deg_hbm, out_hbm,
             qrow, kbuf, vbuf, acc, obuf, albuf, iptrv, degv, idxv,
             semk, semv):
        lane = lax.broadcasted_iota(jnp.int32, (16,), 0)
        wid = lax.axis_index("s") * 2 + lax.axis_index("c")
        base = wid * NPW
        pltpu.sync_copy(iptr2_hbm.at[pl.ds(base, NPW + 16)], iptrv)
        pltpu.sync_copy(deg_hbm.at[pl.ds(base, NPW)], degv)

        def node_body(i, _):
            n_glob = base + i
            start2 = _get_scalar_i32(iptrv, i)
            deg = _get_scalar_i32(degv, i)
            ng = (deg + (grp - 1)) // grp

            pltpu.sync_copy(q_hbm.at[pl.ds(n_glob * hc, hc)], qrow)

            def zero_body(ci, _):
                acc[pl.ds(ci * 16, 16)] = jnp.zeros((16,), jnp.float32)
                return ()
            lax.fori_loop(0, hc // 16, zero_body, (), unroll=8)

            def group_body(g, den_vec):
                eoff = pl.multiple_of(start2 + g * grp, grp)
                pltpu.sync_copy(srcp_hbm.at[pl.ds(eoff, grp)], idxv)
                ck = pltpu.async_copy(k_hbm.at[idxv], kbuf, semk)
                cv = pltpu.async_copy(v_hbm.at[idxv], vbuf, semv)
                ck.wait()
                cv.wait()

                def edge_body(j, den_in):
                    def head_dot(h, avec):
                        qoff = h * c

                        def chunk(cc, a16):
                            return a16 + (qrow[pl.ds(qoff + cc * 16, 16)] *
                                          kbuf[j, pl.ds(qoff + cc * 16, 16)])
                        a16 = lax.fori_loop(
                            0, nch, chunk, jnp.zeros((16,), jnp.float32),
                            unroll=8)
                        s = _lane_sum_splat(a16) * rs
                        return jnp.where(lane == h, s, avec)

                    alpha = lax.fori_loop(
                        0, H, head_dot, jnp.full((16,), -1e30, jnp.float32),
                        unroll=2)
                    ex = jnp.exp(alpha)
                    validf = jnp.where((g * grp + j) < deg, 1.0, 0.0)
                    ex = jnp.where(lane < H, ex, 0.0) * validf

                    def head_acc(h, _):
                        exh = _lane_sum_splat(jnp.where(lane == h, ex, 0.0))
                        aoff = h * c

                        def chunk2(cc, _):
                            plsc.addupdate(
                                acc.at[pl.ds(aoff + cc * 16, 16)],
                                exh * vbuf[j, pl.ds(aoff + cc * 16, 16)])
                            return ()
                        lax.fori_loop(0, nch, chunk2, (), unroll=8)
                        return ()
                    lax.fori_loop(0, H, head_acc, (), unroll=2)
                    return den_in + ex

                return lax.fori_loop(0, grp, edge_body, den_vec, unroll=2)

            den = lax.fori_loop(0, ng, group_body,
                                jnp.zeros((16,), jnp.float32))
            inv = jnp.where(den > 0.0, (1.0 / H) / den, 0.0)

            def out_chunk(cc, _):
                o16 = jnp.zeros((16,), jnp.float32)
                for h in range(H):
                    invh = _lane_sum_splat(jnp.where(lane == h, inv, 0.0))
                    o16 = o16 + invh * acc[pl.ds(h * c + cc * 16, 16)]
                obuf[pl.ds(cc * 16, 16)] = o16
                return ()
            lax.fori_loop(0, nch, out_chunk, (), unroll=4)
            pltpu.sync_copy(obuf, out_hbm.at[pl.ds(n_glob * c, c)])
            return ()

        lax.fori_loop(0, NPW, node_body, ())

    mesh = plsc.VectorSubcoreMesh(core_axis_name="c", subcore_axis_name="s")
    return pl.kernel(
        body,
        out_type=jax.ShapeDtypeStruct((N_PAD * c,), jnp.float32),
        mesh=mesh,
        scratch_types=[
            pltpu.VMEM((hc,), jnp.float32),
            pltpu.VMEM((grp, hc), jnp.float32),
            pltpu.VMEM((grp, hc), jnp.float32),
            pltpu.VMEM((hc,), jnp.float32),
            pltpu.VMEM((c,), jnp.float32),
            pltpu.VMEM((grp * 16,), jnp.float32),
            pltpu.VMEM((NPW + 16,), jnp.int32),
            pltpu.VMEM((NPW,), jnp.int32),
            pltpu.VMEM((grp,), jnp.int32),
            pltpu.SemaphoreType.DMA,
            pltpu.SemaphoreType.DMA,
        ],
    )


_sc_edge_1 = _make_sc_edge_kernel(H * C1, C1, 8)
_sc_edge_2 = _make_sc_edge_kernel(H * C2, C2, 16)


def _pool_body(h2_ref, s2_ref, batch_ref, wg_ref, wf_ref, bf_ref, o_ref):
    h3 = h2_ref[...] + s2_ref[...]
    h3 = jnp.where(h3 > 0, h3, jnp.exp(jnp.minimum(h3, 0.0)) - 1.0)
    gate = jnp.dot(h3, wg_ref[...], preferred_element_type=jnp.float32)[:, 0]
    ge = jnp.exp(gate)
    gid = lax.broadcasted_iota(jnp.int32, (G, N_PAD), 0)
    onehot = jnp.where(batch_ref[...][None, :] == gid, ge[None, :], 0.0)
    den = jnp.sum(onehot, axis=1)
    pooled = jnp.dot(onehot, h3, preferred_element_type=jnp.float32)
    g = pooled / (den[:, None] + 1e-16)
    o_ref[...] = jnp.dot(g, wf_ref[...],
                         preferred_element_type=jnp.float32) + bf_ref[...][None, :]


def _pool(h2, s2, batch_pad, wg, wf, bf):
    return pl.pallas_call(
        _pool_body,
        out_shape=jax.ShapeDtypeStruct((G, OUT), jnp.float32),
    )(h2, s2, batch_pad, wg, wf, bf)


def _elu_add_body(a_ref, b_ref, o_ref):
    s = a_ref[...] + b_ref[...]
    o_ref[...] = jnp.where(s > 0, s, jnp.exp(jnp.minimum(s, 0.0)) - 1.0)


def _elu_add(a, b):
    n, d = a.shape
    return pl.pallas_call(
        _elu_add_body,
        grid=(n // 1024,),
        in_specs=[pl.BlockSpec((1024, d), lambda i: (i, 0)),
                  pl.BlockSpec((1024, d), lambda i: (i, 0))],
        out_specs=pl.BlockSpec((1024, d), lambda i: (i, 0)),
        out_shape=jax.ShapeDtypeStruct((n, d), jnp.float32),
    )(a, b)


def kernel(x, edge_index, batch, Wq1, bq1, Wk1, bk1, Wv1, bv1, Ws1, bs1,
           Wq2, bq2, Wk2, bk2, Wv2, bv2, Ws2, bs2, Wg, bg, Wf, bf):
    src = edge_index[0].astype(jnp.int32)
    dst = edge_index[1].astype(jnp.int32)

    order = jnp.argsort(dst)
    dsts = dst[order]
    srcs = src[order]
    indptr = jnp.searchsorted(dsts, jnp.arange(N_PAD + 1, dtype=jnp.int32),
                              ).astype(jnp.int32)
    deg = jnp.diff(indptr)
    degp = ((deg + 15) // 16) * 16
    iptr2 = jnp.concatenate([jnp.zeros((1,), jnp.int32),
                             jnp.cumsum(degp, dtype=jnp.int32)])
    pos = iptr2[dsts] + (jnp.arange(E, dtype=jnp.int32) - indptr[dsts])
    srcp = jnp.zeros((E2_PAD,), jnp.int32).at[pos].set(srcs)
    iptr2_pad = jnp.concatenate([iptr2,
                                 jnp.zeros((32,), jnp.int32)])
    deg_pad = jnp.concatenate([deg, jnp.zeros((32,), jnp.int32)])

    xp = jnp.pad(x, ((0, N_PAD - N), (0, 0)))

    q1 = _matmul_bias(xp, Wq1, bq1).reshape(-1)
    k1 = _matmul_bias(xp, Wk1, bk1)
    v1 = _matmul_bias(xp, Wv1, bv1)
    s1 = _matmul_bias(xp, Ws1, bs1)
    conv1 = _sc_edge_1(q1, k1, v1, srcp, iptr2_pad, deg_pad
                       ).reshape(N_PAD, C1)
    h1 = _elu_add(conv1, s1)

    q2 = _matmul_bias(h1, Wq2, bq2).reshape(-1)
    k2 = _matmul_bias(h1, Wk2, bk2)
    v2 = _matmul_bias(h1, Wv2, bv2)
    s2 = _matmul_bias(h1, Ws2, bs2)
    conv2 = _sc_edge_2(q2, k2, v2, srcp, iptr2_pad, deg_pad
                       ).reshape(N_PAD, C2)

    batch_pad = jnp.pad(batch.astype(jnp.int32), (0, N_PAD - N),
                        constant_values=G)
    wg_pad = jnp.pad(Wg + 0.0, ((0, 0), (0, 127)))
    wf_pad = Wf
    return _pool(conv2, s2, batch_pad, wg_pad, wf_pad, bf)


# speedup vs baseline: 1.1189x; 1.1189x over previous
import functools
import jax
import jax.numpy as jnp
from jax import lax
from jax.experimental import pallas as pl
from jax.experimental.pallas import tpu as pltpu, tpu_sc as plsc

N = 10000
E = 160000
D = 128
H = 8
C1 = 512
C2 = 256
G = 64
OUT = 10

NW = 32
NPW = 320
N_PAD = NW * NPW
E2_PAD = E + 16 * N_PAD + 64


def _mm_body(x_ref, w_ref, b_ref, o_ref):
    o_ref[...] = jnp.dot(x_ref[...], w_ref[...],
                         preferred_element_type=jnp.float32) + b_ref[...][None, :]


def _matmul_bias(x, w, b, block_n=1024, block_c=512):
    n, din = x.shape
    dout = w.shape[1]
    bc = min(block_c, dout)
    grid = (n // block_n, dout // bc)
    return pl.pallas_call(
        _mm_body,
        grid=grid,
        in_specs=[
            pl.BlockSpec((block_n, din), lambda i, j: (i, 0)),
            pl.BlockSpec((din, bc), lambda i, j: (0, j)),
            pl.BlockSpec((bc,), lambda i, j: (j,)),
        ],
        out_specs=pl.BlockSpec((block_n, bc), lambda i, j: (i, j)),
        out_shape=jax.ShapeDtypeStruct((n, dout), jnp.float32),
    )(x, w, b)


def _perm(x, idx):
    dnums = lax.GatherDimensionNumbers(
        offset_dims=(), collapsed_slice_dims=(0,), start_index_map=(0,))
    return lax.gather(x, idx[:, None], dnums, (1,),
                      mode=lax.GatherScatterMode.PROMISE_IN_BOUNDS)


def _lane_sum_splat(x):
    lane = lax.broadcasted_iota(jnp.int32, (16,), 0)
    for sh in (8, 4, 2, 1):
        x = x + _perm(x, lane ^ sh)
    return x


def _get_scalar_i32(ref, i):
    g = (i // 16) * 16
    vec = ref[pl.ds(g, 16)].astype(jnp.float32)
    lane = lax.broadcasted_iota(jnp.int32, (16,), 0)
    s = _lane_sum_splat(jnp.where(lane == (i - g), vec, 0.0))
    return jnp.squeeze(lax.slice(s, (0,), (1,))).astype(jnp.int32)


def _make_sc_edge_kernel(hc, c, grp):
    rs = 1.0 / (c ** 0.5)
    nch = c // 16

    def body(q_hbm, k_hbm, v_hbm, srcp_hbm, iptr2_hbm, deg_hbm, out_hbm,
             qrow, kbuf, vbuf, acc, obuf, iptrv, degv, idxv, semk, semv):
        lane = lax.broadcasted_iota(jnp.int32, (16,), 0)
        wid = lax.axis_index("s") * 2 + lax.axis_index("c")
        base = wid * NPW
        pltpu.sync_copy(iptr2_hbm.at[pl.ds(base, NPW + 16)], iptrv)
        pltpu.sync_copy(deg_hbm.at[pl.ds(base, NPW)], degv)

        def node_body(i, _):
            n_glob = base + i
            start2 = _get_scalar_i32(iptrv, i)
            deg = _get_scalar_i32(degv, i)
            ng = (deg + (grp - 1)) // grp

            pltpu.sync_copy(q_hbm.at[pl.ds(n_glob * hc, hc)], qrow)

            def zero_body(ci, _):
                acc[pl.ds(ci * 16, 16)] = jnp.zeros((16,), jnp.float32)
                return ()
            lax.fori_loop(0, hc // 16, zero_body, (), unroll=8)

            def group_body(g, den_vec):
                eoff = pl.multiple_of(start2 + g * grp, grp)
                pltpu.sync_copy(srcp_hbm.at[pl.ds(eoff, grp)], idxv)
                ck = pltpu.async_copy(k_hbm.at[idxv], kbuf, semk)
                cv = pltpu.async_copy(v_hbm.at[idxv], vbuf, semv)
                ck.wait()
                cv.wait()

                def edge_body(j, den_in):
                    def head_dot(h, avec):
                        qoff = h * c

                        def chunk(cc, a16):
                            return a16 + (qrow[pl.ds(qoff + cc * 16, 16)] *
                                          kbuf[j, pl.ds(qoff + cc * 16, 16)])
                        a16 = lax.fori_loop(
                            0, nch, chunk, jnp.zeros((16,), jnp.float32),
                            unroll=8)
                        s = _lane_sum_splat(a16) * rs
                        return jnp.where(lane == h, s, avec)

                    alpha = lax.fori_loop(
                        0, H, head_dot, jnp.full((16,), -1e30, jnp.float32))
                    ex = jnp.exp(alpha)
                    validf = jnp.where((g * grp + j) < deg, 1.0, 0.0)
                    ex = jnp.where(lane < H, ex, 0.0) * validf

                    def head_acc(h, _):
                        exh = _lane_sum_splat(jnp.where(lane == h, ex, 0.0))
                        aoff = h * c

                        def chunk2(cc, _):
                            plsc.addupdate(
                                acc.at[pl.ds(aoff + cc * 16, 16)],
                                exh * vbuf[j, pl.ds(aoff + cc * 16, 16)])
                            return ()
                        lax.fori_loop(0, nch, chunk2, (), unroll=8)
                        return ()
                    lax.fori_loop(0, H, head_acc, ())
                    return den_in + ex

                return lax.fori_loop(0, grp, edge_body, den_vec)

            den = lax.fori_loop(0, ng, group_body,
                                jnp.zeros((16,), jnp.float32))
            inv = jnp.where(den > 0.0, (1.0 / H) / den, 0.0)

            def out_chunk(cc, _):
                o16 = jnp.zeros((16,), jnp.float32)
                for h in range(H):
                    invh = _lane_sum_splat(jnp.where(lane == h, inv, 0.0))
                    o16 = o16 + invh * acc[pl.ds(h * c + cc * 16, 16)]
                obuf[pl.ds(cc * 16, 16)] = o16
                return ()
            lax.fori_loop(0, nch, out_chunk, (), unroll=4)
            pltpu.sync_copy(obuf, out_hbm.at[pl.ds(n_glob * c, c)])
            return ()

        lax.fori_loop(0, NPW, node_body, ())

    mesh = plsc.VectorSubcoreMesh(core_axis_name="c", subcore_axis_name="s")
    return pl.kernel(
        body,
        out_type=jax.ShapeDtypeStruct((N_PAD * c,), jnp.float32),
        mesh=mesh,
        scratch_types=[
            pltpu.VMEM((hc,), jnp.float32),
            pltpu.VMEM((grp, hc), jnp.float32),
            pltpu.VMEM((grp, hc), jnp.float32),
            pltpu.VMEM((hc,), jnp.float32),
            pltpu.VMEM((c,), jnp.float32),
            pltpu.VMEM((NPW + 16,), jnp.int32),
            pltpu.VMEM((NPW,), jnp.int32),
            pltpu.VMEM((grp,), jnp.int32),
            pltpu.SemaphoreType.DMA,
            pltpu.SemaphoreType.DMA,
        ],
    )


_sc_edge_1 = _make_sc_edge_kernel(H * C1, C1, 8)
_sc_edge_2 = _make_sc_edge_kernel(H * C2, C2, 16)


def _pool_body(h2_ref, s2_ref, batch_ref, wg_ref, wf_ref, bf_ref, o_ref):
    h3 = h2_ref[...] + s2_ref[...]
    h3 = jnp.where(h3 > 0, h3, jnp.exp(jnp.minimum(h3, 0.0)) - 1.0)
    gate = jnp.dot(h3, wg_ref[...], preferred_element_type=jnp.float32)[:, 0]
    ge = jnp.exp(gate)
    gid = lax.broadcasted_iota(jnp.int32, (G, N_PAD), 0)
    onehot = jnp.where(batch_ref[...][None, :] == gid, ge[None, :], 0.0)
    den = jnp.sum(onehot, axis=1)
    pooled = jnp.dot(onehot, h3, preferred_element_type=jnp.float32)
    g = pooled / (den[:, None] + 1e-16)
    o_ref[...] = jnp.dot(g, wf_ref[...],
                         preferred_element_type=jnp.float32) + bf_ref[...][None, :]


def _pool(h2, s2, batch_pad, wg, wf, bf):
    return pl.pallas_call(
        _pool_body,
        out_shape=jax.ShapeDtypeStruct((G, OUT), jnp.float32),
    )(h2, s2, batch_pad, wg, wf, bf)


def _elu_add_body(a_ref, b_ref, o_ref):
    s = a_ref[...] + b_ref[...]
    o_ref[...] = jnp.where(s > 0, s, jnp.exp(jnp.minimum(s, 0.0)) - 1.0)


def _elu_add(a, b):
    n, d = a.shape
    return pl.pallas_call(
        _elu_add_body,
        grid=(n // 1024,),
        in_specs=[pl.BlockSpec((1024, d), lambda i: (i, 0)),
                  pl.BlockSpec((1024, d), lambda i: (i, 0))],
        out_specs=pl.BlockSpec((1024, d), lambda i: (i, 0)),
        out_shape=jax.ShapeDtypeStruct((n, d), jnp.float32),
    )(a, b)


def kernel(x, edge_index, batch, Wq1, bq1, Wk1, bk1, Wv1, bv1, Ws1, bs1,
           Wq2, bq2, Wk2, bk2, Wv2, bv2, Ws2, bs2, Wg, bg, Wf, bf):
    src = edge_index[0].astype(jnp.int32)
    dst = edge_index[1].astype(jnp.int32)

    order = jnp.argsort(dst)
    dsts = dst[order]
    srcs = src[order]
    indptr = jnp.searchsorted(dsts, jnp.arange(N_PAD + 1, dtype=jnp.int32),
                              ).astype(jnp.int32)
    deg = jnp.diff(indptr)
    degp = ((deg + 15) // 16) * 16
    iptr2 = jnp.concatenate([jnp.zeros((1,), jnp.int32),
                             jnp.cumsum(degp, dtype=jnp.int32)])
    pos = iptr2[dsts] + (jnp.arange(E, dtype=jnp.int32) - indptr[dsts])
    srcp = jnp.zeros((E2_PAD,), jnp.int32).at[pos].set(srcs)
    iptr2_pad = jnp.concatenate([iptr2,
                                 jnp.zeros((32,), jnp.int32)])
    deg_pad = jnp.concatenate([deg, jnp.zeros((32,), jnp.int32)])

    xp = jnp.pad(x, ((0, N_PAD - N), (0, 0)))

    q1 = _matmul_bias(xp, Wq1, bq1).reshape(-1)
    k1 = _matmul_bias(xp, Wk1, bk1)
    v1 = _matmul_bias(xp, Wv1, bv1)
    s1 = _matmul_bias(xp, Ws1, bs1)
    conv1 = _sc_edge_1(q1, k1, v1, srcp, iptr2_pad, deg_pad
                       ).reshape(N_PAD, C1)
    h1 = _elu_add(conv1, s1)

    q2 = _matmul_bias(h1, Wq2, bq2).reshape(-1)
    k2 = _matmul_bias(h1, Wk2, bk2)
    v2 = _matmul_bias(h1, Wv2, bv2)
    s2 = _matmul_bias(h1, Ws2, bs2)
    conv2 = _sc_edge_2(q2, k2, v2, srcp, iptr2_pad, deg_pad
                       ).reshape(N_PAD, C2)

    batch_pad = jnp.pad(batch.astype(jnp.int32), (0, N_PAD - N),
                        constant_values=G)
    wg_pad = jnp.pad(Wg + 0.0, ((0, 0), (0, 127)))
    wf_pad = Wf
    return _pool(conv2, s2, batch_pad, wg_pad, wf_pad, bf)
